# baseline (speedup 1.0000x reference)
import functools

import jax
import jax.numpy as jnp
from jax import lax
from jax.experimental import pallas as pl
from jax.experimental.pallas import tpu as pltpu
from jax.experimental.pallas import tpu_sc as plsc

_N_E = 8192
_E_DIM = 32
_BETA = 0.25
_N_PTS = 8192

_RB = 1024
_CB = 2048
_N_RB = _N_PTS // _RB
_N_CB = _N_E // _CB
_NT = _CB // 128


def _argmin_body(zf_ref, e_ref, idx_ref, loss_ref, md_ref, acc_ref):
    j = pl.program_id(1)
    zf = zf_ref[...]
    e = e_ref[...]
    sz = jnp.sum(zf * zf, axis=1, keepdims=True)
    se = jnp.sum(e * e, axis=1)
    dot = jax.lax.dot_general(
        zf.astype(jnp.bfloat16), e.astype(jnp.bfloat16),
        (((1,), (1,)), ((), ())),
        preferred_element_type=jnp.float32)
    dist = (sz + se[None, :]) - 2.0 * dot
    tiles = [dist[:, k * 128:(k + 1) * 128] for k in range(_NT)]
    m1 = tiles[0]
    for k in range(1, _NT):
        m1 = jnp.minimum(m1, tiles[k])
    big = jnp.int32(2**30)
    it1 = jnp.where(tiles[0] == m1, jnp.int32(0), big)
    for k in range(1, _NT):
        it1 = jnp.minimum(it1, jnp.where(tiles[k] == m1, jnp.int32(k), big))
    lane = jax.lax.broadcasted_iota(jnp.int32, (_RB, 128), 1)
    cand = it1 * 128 + lane
    wmin = jnp.min(m1, axis=1)
    sel = jnp.where(m1 == wmin[:, None], cand, big)
    widx = jnp.min(sel, axis=1) + j * _CB
    wmin_q = wmin.astype(jnp.bfloat16).astype(jnp.float32)

    @pl.when(j == 0)
    def _():
        md_ref[...] = wmin_q
        idx_ref[...] = widx

    @pl.when(j > 0)
    def _():
        cur = md_ref[...]
        repl = wmin < cur
        md_ref[...] = jnp.where(repl, wmin_q, cur)
        idx_ref[...] = jnp.where(repl, widx, idx_ref[...])

    @pl.when(j == _N_CB - 1)
    def _():
        i = pl.program_id(0)

        @pl.when(i == 0)
        def _():
            acc_ref[0] = 0.0

        acc_ref[0] += jnp.sum(md_ref[...])

        @pl.when(i == _N_RB - 1)
        def _():
            loss_ref[0] = acc_ref[0] * (_BETA / (_N_PTS * _E_DIM))


def _argmin_search(zf, emb):
    idx, loss = pl.pallas_call(
        _argmin_body,
        grid=(_N_RB, _N_CB),
        in_specs=[
            pl.BlockSpec((_RB, _E_DIM), lambda i, j: (i, 0)),
            pl.BlockSpec((_CB, _E_DIM), lambda i, j: (j, 0)),
        ],
        out_specs=[
            pl.BlockSpec((_RB,), lambda i, j: (i,)),
            pl.BlockSpec(memory_space=pltpu.SMEM),
        ],
        out_shape=[
            jax.ShapeDtypeStruct((_N_PTS,), jnp.int32),
            jax.ShapeDtypeStruct((1,), jnp.float32),
        ],
        scratch_shapes=[
            pltpu.VMEM((_RB,), jnp.float32),
            pltpu.SMEM((1,), jnp.float32),
        ],
    )(zf, emb)
    return idx, loss[0]


_NS = 16
_RPT = _N_PTS // _NS
_NCH = _RPT // 128


def _sc_body(emb_hbm, idx2_hbm, zf_hbm, cs_hbm, zq_hbm, counts_hbm,
             idx_v, rows_v, zp_v, ones_v, counts_sp, sem):
    cid = lax.axis_index("c")
    sid = lax.axis_index("s")

    @pl.when(cid == 0)
    def _gather():
        pltpu.sync_copy(idx2_hbm.at[pl.ds(sid * _NCH, _NCH)], idx_v)
        for c in range(_NCH):
            rbase = sid * _RPT + c * 128
            pltpu.async_copy(emb_hbm.at[idx_v.at[c]], rows_v, sem).wait()
            pltpu.sync_copy(zf_hbm.at[pl.ds(rbase, 128)], zp_v)

            def _st(i, carry):
                r = i // 2
                col = (i % 2) * 16
                zr = zp_v[r, pl.ds(col, 16)]
                g = rows_v[r, pl.ds(col, 16)]
                rows_v[r, pl.ds(col, 16)] = zr + (g - zr)
                return carry

            lax.fori_loop(0, 256, _st, 0)
            pltpu.sync_copy(rows_v, zq_hbm.at[pl.ds(rbase, 128)])

    @pl.when(cid == 1)
    def _hist():
        @pl.when(sid == 0)
        def _():
            pltpu.sync_copy(cs_hbm, counts_sp)
        pltpu.sync_copy(idx2_hbm.at[pl.ds(sid * _NCH, _NCH)], idx_v)
        for k in range(8):
            ones_v[pl.ds(k * 16, 16)] = jnp.full((16,), 1.0, jnp.float32)
        plsc.subcore_barrier()
        for c in range(_NCH):
            pltpu.sync_copy(ones_v, counts_sp.at[idx_v.at[c]], add=True)
        plsc.subcore_barrier()
        pltpu.sync_copy(counts_sp.at[pl.ds(sid * _RPT, _RPT)],
                        counts_hbm.at[pl.ds(sid * _RPT, _RPT)])


@functools.cache
def _sc_gather_hist():
    return pl.kernel(
        _sc_body,
        out_type=[
            jax.ShapeDtypeStruct((_N_PTS, _E_DIM), jnp.float32),
            jax.ShapeDtypeStruct((_N_E,), jnp.float32),
        ],
        mesh=plsc.VectorSubcoreMesh(
            core_axis_name="c", subcore_axis_name="s",
            num_cores=2, num_subcores=_NS),
        compiler_params=pltpu.CompilerParams(use_tc_tiling_on_sc=False),
        scratch_types=[
            pltpu.VMEM((_NCH, 128), jnp.int32),
            pltpu.VMEM((128, _E_DIM), jnp.float32),
            pltpu.VMEM((128, _E_DIM), jnp.float32),
            pltpu.VMEM((128,), jnp.float32),
            pltpu.VMEM_SHARED((_N_E,), jnp.float32),
            pltpu.SemaphoreType.DMA,
        ],
    )


def kernel(z, embedding, cluster_size, cluster_sum):
    zp = jnp.transpose(z, (0, 2, 3, 1))
    zf = zp.reshape(-1, _E_DIM)
    idx, loss = _argmin_search(zf, embedding)
    idx2 = idx.reshape(_N_PTS // 128, 128)
    zq_st, counts = _sc_gather_hist()(embedding, idx2, zf, cluster_size)
    z_q_out = jnp.transpose(zq_st.reshape(zp.shape), (0, 3, 1, 2))
    return (z_q_out, loss, counts)


# speedup vs baseline: 1.0694x; 1.0694x over previous
import functools

import jax
import jax.numpy as jnp
from jax import lax
from jax.experimental import pallas as pl
from jax.experimental.pallas import tpu as pltpu
from jax.experimental.pallas import tpu_sc as plsc

_N_E = 8192
_E_DIM = 32
_BETA = 0.25
_N_PTS = 8192

_RB = 1024
_CB = 2048
_N_RB = _N_PTS // _RB
_N_CB = _N_E // _CB
_NT = _CB // 128


def _argmin_body(zf_ref, e_ref, idx_ref, loss_ref, md_ref, acc_ref):
    j = pl.program_id(1)
    zf = zf_ref[...]
    e = e_ref[...]
    sz = jnp.sum(zf * zf, axis=1, keepdims=True)
    se = jnp.sum(e * e, axis=1)
    dot = jax.lax.dot_general(
        zf.astype(jnp.bfloat16), e.astype(jnp.bfloat16),
        (((1,), (1,)), ((), ())),
        preferred_element_type=jnp.float32)
    big = jnp.int32(2**30)

    def tile_dist(k):
        sek = se[k * 128:(k + 1) * 128]
        dotk = dot[:, k * 128:(k + 1) * 128]
        return (sz + sek[None, :]) - 2.0 * dotk

    m1 = tile_dist(0)
    it1 = jnp.zeros((_RB, 128), jnp.int32)
    for k in range(1, _NT):
        dk = tile_dist(k)
        lt = dk < m1
        m1 = jnp.where(lt, dk, m1)
        it1 = jnp.where(lt, jnp.int32(k), it1)
    lane = jax.lax.broadcasted_iota(jnp.int32, (_RB, 128), 1)
    cand = it1 * 128 + lane
    wmin = jnp.min(m1, axis=1)
    sel = jnp.where(m1 == wmin[:, None], cand, big)
    widx = jnp.min(sel, axis=1) + j * _CB
    wmin_q = wmin.astype(jnp.bfloat16).astype(jnp.float32)

    @pl.when(j == 0)
    def _():
        md_ref[...] = wmin_q
        idx_ref[...] = widx

    @pl.when(j > 0)
    def _():
        cur = md_ref[...]
        repl = wmin < cur
        md_ref[...] = jnp.where(repl, wmin_q, cur)
        idx_ref[...] = jnp.where(repl, widx, idx_ref[...])

    @pl.when(j == _N_CB - 1)
    def _():
        i = pl.program_id(0)

        @pl.when(i == 0)
        def _():
            acc_ref[0] = 0.0

        acc_ref[0] += jnp.sum(md_ref[...])

        @pl.when(i == _N_RB - 1)
        def _():
            loss_ref[0] = acc_ref[0] * (_BETA / (_N_PTS * _E_DIM))


def _argmin_search(zf, emb):
    idx, loss = pl.pallas_call(
        _argmin_body,
        grid=(_N_RB, _N_CB),
        in_specs=[
            pl.BlockSpec((_RB, _E_DIM), lambda i, j: (i, 0)),
            pl.BlockSpec((_CB, _E_DIM), lambda i, j: (j, 0)),
        ],
        out_specs=[
            pl.BlockSpec((_RB,), lambda i, j: (i,)),
            pl.BlockSpec(memory_space=pltpu.SMEM),
        ],
        out_shape=[
            jax.ShapeDtypeStruct((_N_PTS,), jnp.int32),
            jax.ShapeDtypeStruct((1,), jnp.float32),
        ],
        scratch_shapes=[
            pltpu.VMEM((_RB,), jnp.float32),
            pltpu.SMEM((1,), jnp.float32),
        ],
    )(zf, emb)
    return idx, loss[0]


_NS = 16
_RPT = _N_PTS // _NS
_NCH = _RPT // 128


def _sc_body(emb_hbm, idx2_hbm, zf_hbm, cs_hbm, zq_hbm, counts_hbm,
             idx_v, rows_v, zp_v, ones_v, counts_sp, sem):
    cid = lax.axis_index("c")
    sid = lax.axis_index("s")

    @pl.when(cid == 0)
    def _gather():
        pltpu.sync_copy(idx2_hbm.at[pl.ds(sid * _NCH, _NCH)], idx_v)
        for c in range(_NCH):
            rbase = sid * _RPT + c * 128
            pltpu.async_copy(emb_hbm.at[idx_v.at[c]], rows_v, sem).wait()
            pltpu.sync_copy(zf_hbm.at[pl.ds(rbase, 128)], zp_v)

            def _st(i, carry):
                r = i // 2
                col = (i % 2) * 16
                zr = zp_v[r, pl.ds(col, 16)]
                g = rows_v[r, pl.ds(col, 16)]
                rows_v[r, pl.ds(col, 16)] = zr + (g - zr)
                return carry

            lax.fori_loop(0, 256, _st, 0)
            pltpu.sync_copy(rows_v, zq_hbm.at[pl.ds(rbase, 128)])

    @pl.when(cid == 1)
    def _hist():
        @pl.when(sid == 0)
        def _():
            pltpu.sync_copy(cs_hbm, counts_sp)
        pltpu.sync_copy(idx2_hbm.at[pl.ds(sid * _NCH, _NCH)], idx_v)
        for k in range(8):
            ones_v[pl.ds(k * 16, 16)] = jnp.full((16,), 1.0, jnp.float32)
        plsc.subcore_barrier()
        for c in range(_NCH):
            pltpu.sync_copy(ones_v, counts_sp.at[idx_v.at[c]], add=True)
        plsc.subcore_barrier()
        pltpu.sync_copy(counts_sp.at[pl.ds(sid * _RPT, _RPT)],
                        counts_hbm.at[pl.ds(sid * _RPT, _RPT)])


@functools.cache
def _sc_gather_hist():
    return pl.kernel(
        _sc_body,
        out_type=[
            jax.ShapeDtypeStruct((_N_PTS, _E_DIM), jnp.float32),
            jax.ShapeDtypeStruct((_N_E,), jnp.float32),
        ],
        mesh=plsc.VectorSubcoreMesh(
            core_axis_name="c", subcore_axis_name="s",
            num_cores=2, num_subcores=_NS),
        compiler_params=pltpu.CompilerParams(use_tc_tiling_on_sc=False),
        scratch_types=[
            pltpu.VMEM((_NCH, 128), jnp.int32),
            pltpu.VMEM((128, _E_DIM), jnp.float32),
            pltpu.VMEM((128, _E_DIM), jnp.float32),
            pltpu.VMEM((128,), jnp.float32),
            pltpu.VMEM_SHARED((_N_E,), jnp.float32),
            pltpu.SemaphoreType.DMA,
        ],
    )


def kernel(z, embedding, cluster_size, cluster_sum):
    zp = jnp.transpose(z, (0, 2, 3, 1))
    zf = zp.reshape(-1, _E_DIM)
    idx, loss = _argmin_search(zf, embedding)
    idx2 = idx.reshape(_N_PTS // 128, 128)
    zq_st, counts = _sc_gather_hist()(embedding, idx2, zf, cluster_size)
    z_q_out = jnp.transpose(zq_st.reshape(zp.shape), (0, 3, 1, 2))
    return (z_q_out, loss, counts)


# speedup vs baseline: 1.1295x; 1.0562x over previous
import functools

import jax
import jax.numpy as jnp
from jax import lax
from jax.experimental import pallas as pl
from jax.experimental.pallas import tpu as pltpu
from jax.experimental.pallas import tpu_sc as plsc

_N_E = 8192
_E_DIM = 32
_BETA = 0.25
_N_PTS = 8192

_RB = 2048
_CB = 2048
_N_RB = _N_PTS // _RB
_N_CB = _N_E // _CB
_NT = _CB // 128


def _argmin_body(zf_ref, e_ref, idx_ref, loss_ref, md_ref, acc_ref):
    j = pl.program_id(1)
    zf = zf_ref[...]
    e = e_ref[...]
    sz = jnp.sum(zf * zf, axis=1, keepdims=True)
    se = jnp.sum(e * e, axis=1)
    dot = jax.lax.dot_general(
        zf.astype(jnp.bfloat16), e.astype(jnp.bfloat16),
        (((1,), (1,)), ((), ())),
        preferred_element_type=jnp.float32)
    big = jnp.int32(2**30)

    def tile_dist(k):
        sek = se[k * 128:(k + 1) * 128]
        dotk = dot[:, k * 128:(k + 1) * 128]
        return (sz + sek[None, :]) - 2.0 * dotk

    m1 = tile_dist(0)
    it1 = jnp.zeros((_RB, 128), jnp.int32)
    for k in range(1, _NT):
        dk = tile_dist(k)
        lt = dk < m1
        m1 = jnp.where(lt, dk, m1)
        it1 = jnp.where(lt, jnp.int32(k), it1)
    lane = jax.lax.broadcasted_iota(jnp.int32, (_RB, 128), 1)
    cand = it1 * 128 + lane
    wmin = jnp.min(m1, axis=1)
    sel = jnp.where(m1 == wmin[:, None], cand, big)
    widx = jnp.min(sel, axis=1) + j * _CB
    wmin_q = wmin.astype(jnp.bfloat16).astype(jnp.float32)

    @pl.when(j == 0)
    def _():
        md_ref[...] = wmin_q
        idx_ref[...] = widx

    @pl.when(j > 0)
    def _():
        cur = md_ref[...]
        repl = wmin < cur
        md_ref[...] = jnp.where(repl, wmin_q, cur)
        idx_ref[...] = jnp.where(repl, widx, idx_ref[...])

    @pl.when(j == _N_CB - 1)
    def _():
        i = pl.program_id(0)

        @pl.when(i == 0)
        def _():
            acc_ref[0] = 0.0

        acc_ref[0] += jnp.sum(md_ref[...])

        @pl.when(i == _N_RB - 1)
        def _():
            loss_ref[0] = acc_ref[0] * (_BETA / (_N_PTS * _E_DIM))


def _argmin_search(zf, emb):
    idx, loss = pl.pallas_call(
        _argmin_body,
        grid=(_N_RB, _N_CB),
        in_specs=[
            pl.BlockSpec((_RB, _E_DIM), lambda i, j: (i, 0)),
            pl.BlockSpec((_CB, _E_DIM), lambda i, j: (j, 0)),
        ],
        out_specs=[
            pl.BlockSpec((_RB,), lambda i, j: (i,)),
            pl.BlockSpec(memory_space=pltpu.SMEM),
        ],
        out_shape=[
            jax.ShapeDtypeStruct((_N_PTS,), jnp.int32),
            jax.ShapeDtypeStruct((1,), jnp.float32),
        ],
        scratch_shapes=[
            pltpu.VMEM((_RB,), jnp.float32),
            pltpu.SMEM((1,), jnp.float32),
        ],
    )(zf, emb)
    return idx, loss[0]


_NS = 16
_RPT = _N_PTS // _NS
_NCH = _RPT // 128


def _sc_body(emb_hbm, idx2_hbm, zf_hbm, cs_hbm, zq_hbm, counts_hbm,
             idx_v, rows_v, zp_v, ones_v, counts_sp, sem):
    cid = lax.axis_index("c")
    sid = lax.axis_index("s")

    @pl.when(cid == 0)
    def _gather():
        pltpu.sync_copy(idx2_hbm.at[pl.ds(sid * _NCH, _NCH)], idx_v)
        for c in range(_NCH):
            rbase = sid * _RPT + c * 128
            pltpu.async_copy(emb_hbm.at[idx_v.at[c]], rows_v, sem).wait()
            pltpu.sync_copy(zf_hbm.at[pl.ds(rbase, 128)], zp_v)

            def _st(i, carry):
                r = i // 2
                col = (i % 2) * 16
                zr = zp_v[r, pl.ds(col, 16)]
                g = rows_v[r, pl.ds(col, 16)]
                rows_v[r, pl.ds(col, 16)] = zr + (g - zr)
                return carry

            lax.fori_loop(0, 256, _st, 0)
            pltpu.sync_copy(rows_v, zq_hbm.at[pl.ds(rbase, 128)])

    @pl.when(cid == 1)
    def _hist():
        @pl.when(sid == 0)
        def _():
            pltpu.sync_copy(cs_hbm, counts_sp)
        pltpu.sync_copy(idx2_hbm.at[pl.ds(sid * _NCH, _NCH)], idx_v)
        for k in range(8):
            ones_v[pl.ds(k * 16, 16)] = jnp.full((16,), 1.0, jnp.float32)
        plsc.subcore_barrier()
        for c in range(_NCH):
            pltpu.sync_copy(ones_v, counts_sp.at[idx_v.at[c]], add=True)
        plsc.subcore_barrier()
        pltpu.sync_copy(counts_sp.at[pl.ds(sid * _RPT, _RPT)],
                        counts_hbm.at[pl.ds(sid * _RPT, _RPT)])


@functools.cache
def _sc_gather_hist():
    return pl.kernel(
        _sc_body,
        out_type=[
            jax.ShapeDtypeStruct((_N_PTS, _E_DIM), jnp.float32),
            jax.ShapeDtypeStruct((_N_E,), jnp.float32),
        ],
        mesh=plsc.VectorSubcoreMesh(
            core_axis_name="c", subcore_axis_name="s",
            num_cores=2, num_subcores=_NS),
        compiler_params=pltpu.CompilerParams(use_tc_tiling_on_sc=False),
        scratch_types=[
            pltpu.VMEM((_NCH, 128), jnp.int32),
            pltpu.VMEM((128, _E_DIM), jnp.float32),
            pltpu.VMEM((128, _E_DIM), jnp.float32),
            pltpu.VMEM((128,), jnp.float32),
            pltpu.VMEM_SHARED((_N_E,), jnp.float32),
            pltpu.SemaphoreType.DMA,
        ],
    )


def kernel(z, embedding, cluster_size, cluster_sum):
    zp = jnp.transpose(z, (0, 2, 3, 1))
    zf = zp.reshape(-1, _E_DIM)
    idx, loss = _argmin_search(zf, embedding)
    idx2 = idx.reshape(_N_PTS // 128, 128)
    zq_st, counts = _sc_gather_hist()(embedding, idx2, zf, cluster_size)
    z_q_out = jnp.transpose(zq_st.reshape(zp.shape), (0, 3, 1, 2))
    return (z_q_out, loss, counts)


# speedup vs baseline: 1.1625x; 1.0292x over previous
import functools

import jax
import jax.numpy as jnp
from jax import lax
from jax.experimental import pallas as pl
from jax.experimental.pallas import tpu as pltpu
from jax.experimental.pallas import tpu_sc as plsc

_N_E = 8192
_E_DIM = 32
_BETA = 0.25
_N_PTS = 8192

_RB = 4096
_CB = 2048
_N_RB = _N_PTS // _RB
_N_CB = _N_E // _CB
_NT = _CB // 128


def _argmin_body(zf_ref, e_ref, idx_ref, loss_ref, md_ref, acc_ref):
    j = pl.program_id(1)
    zf = zf_ref[...]
    e = e_ref[...]
    sz = jnp.sum(zf * zf, axis=1, keepdims=True)
    se = jnp.sum(e * e, axis=1)
    dot = jax.lax.dot_general(
        zf.astype(jnp.bfloat16), e.astype(jnp.bfloat16),
        (((1,), (1,)), ((), ())),
        preferred_element_type=jnp.float32)
    big = jnp.int32(2**30)

    def tile_dist(k):
        sek = se[k * 128:(k + 1) * 128]
        dotk = dot[:, k * 128:(k + 1) * 128]
        return (sz + sek[None, :]) - 2.0 * dotk

    m1 = tile_dist(0)
    it1 = jnp.zeros((_RB, 128), jnp.int32)
    for k in range(1, _NT):
        dk = tile_dist(k)
        lt = dk < m1
        m1 = jnp.where(lt, dk, m1)
        it1 = jnp.where(lt, jnp.int32(k), it1)
    lane = jax.lax.broadcasted_iota(jnp.int32, (_RB, 128), 1)
    cand = it1 * 128 + lane
    wmin = jnp.min(m1, axis=1)
    sel = jnp.where(m1 == wmin[:, None], cand, big)
    widx = jnp.min(sel, axis=1) + j * _CB
    wmin_q = wmin.astype(jnp.bfloat16).astype(jnp.float32)

    @pl.when(j == 0)
    def _():
        md_ref[...] = wmin_q
        idx_ref[...] = widx

    @pl.when(j > 0)
    def _():
        cur = md_ref[...]
        repl = wmin < cur
        md_ref[...] = jnp.where(repl, wmin_q, cur)
        idx_ref[...] = jnp.where(repl, widx, idx_ref[...])

    @pl.when(j == _N_CB - 1)
    def _():
        i = pl.program_id(0)

        @pl.when(i == 0)
        def _():
            acc_ref[0] = 0.0

        acc_ref[0] += jnp.sum(md_ref[...])

        @pl.when(i == _N_RB - 1)
        def _():
            loss_ref[0] = acc_ref[0] * (_BETA / (_N_PTS * _E_DIM))


def _argmin_search(zf, emb):
    idx, loss = pl.pallas_call(
        _argmin_body,
        grid=(_N_RB, _N_CB),
        in_specs=[
            pl.BlockSpec((_RB, _E_DIM), lambda i, j: (i, 0)),
            pl.BlockSpec((_CB, _E_DIM), lambda i, j: (j, 0)),
        ],
        out_specs=[
            pl.BlockSpec((_RB,), lambda i, j: (i,)),
            pl.BlockSpec(memory_space=pltpu.SMEM),
        ],
        out_shape=[
            jax.ShapeDtypeStruct((_N_PTS,), jnp.int32),
            jax.ShapeDtypeStruct((1,), jnp.float32),
        ],
        scratch_shapes=[
            pltpu.VMEM((_RB,), jnp.float32),
            pltpu.SMEM((1,), jnp.float32),
        ],
    )(zf, emb)
    return idx, loss[0]


_NS = 16
_RPT = _N_PTS // _NS
_NCH = _RPT // 128


def _sc_body(emb_hbm, idx2_hbm, zf_hbm, cs_hbm, zq_hbm, counts_hbm,
             idx_v, rows_v, zp_v, ones_v, counts_sp, sem):
    cid = lax.axis_index("c")
    sid = lax.axis_index("s")

    @pl.when(cid == 0)
    def _gather():
        pltpu.sync_copy(idx2_hbm.at[pl.ds(sid * _NCH, _NCH)], idx_v)
        for c in range(_NCH):
            rbase = sid * _RPT + c * 128
            pltpu.async_copy(emb_hbm.at[idx_v.at[c]], rows_v, sem).wait()
            pltpu.sync_copy(zf_hbm.at[pl.ds(rbase, 128)], zp_v)

            def _st(i, carry):
                r = i // 2
                col = (i % 2) * 16
                zr = zp_v[r, pl.ds(col, 16)]
                g = rows_v[r, pl.ds(col, 16)]
                rows_v[r, pl.ds(col, 16)] = zr + (g - zr)
                return carry

            lax.fori_loop(0, 256, _st, 0)
            pltpu.sync_copy(rows_v, zq_hbm.at[pl.ds(rbase, 128)])

    @pl.when(cid == 1)
    def _hist():
        @pl.when(sid == 0)
        def _():
            pltpu.sync_copy(cs_hbm, counts_sp)
        pltpu.sync_copy(idx2_hbm.at[pl.ds(sid * _NCH, _NCH)], idx_v)
        for k in range(8):
            ones_v[pl.ds(k * 16, 16)] = jnp.full((16,), 1.0, jnp.float32)
        plsc.subcore_barrier()
        for c in range(_NCH):
            pltpu.sync_copy(ones_v, counts_sp.at[idx_v.at[c]], add=True)
        plsc.subcore_barrier()
        pltpu.sync_copy(counts_sp.at[pl.ds(sid * _RPT, _RPT)],
                        counts_hbm.at[pl.ds(sid * _RPT, _RPT)])


@functools.cache
def _sc_gather_hist():
    return pl.kernel(
        _sc_body,
        out_type=[
            jax.ShapeDtypeStruct((_N_PTS, _E_DIM), jnp.float32),
            jax.ShapeDtypeStruct((_N_E,), jnp.float32),
        ],
        mesh=plsc.VectorSubcoreMesh(
            core_axis_name="c", subcore_axis_name="s",
            num_cores=2, num_subcores=_NS),
        compiler_params=pltpu.CompilerParams(use_tc_tiling_on_sc=False),
        scratch_types=[
            pltpu.VMEM((_NCH, 128), jnp.int32),
            pltpu.VMEM((128, _E_DIM), jnp.float32),
            pltpu.VMEM((128, _E_DIM), jnp.float32),
            pltpu.VMEM((128,), jnp.float32),
            pltpu.VMEM_SHARED((_N_E,), jnp.float32),
            pltpu.SemaphoreType.DMA,
        ],
    )


def kernel(z, embedding, cluster_size, cluster_sum):
    zp = jnp.transpose(z, (0, 2, 3, 1))
    zf = zp.reshape(-1, _E_DIM)
    idx, loss = _argmin_search(zf, embedding)
    idx2 = idx.reshape(_N_PTS // 128, 128)
    zq_st, counts = _sc_gather_hist()(embedding, idx2, zf, cluster_size)
    z_q_out = jnp.transpose(zq_st.reshape(zp.shape), (0, 3, 1, 2))
    return (z_q_out, loss, counts)
